# Initial kernel scaffold; baseline (speedup 1.0000x reference)
import jax
import jax.numpy as jnp
from jax.experimental import pallas as pl


def kernel(x, edge_index, m1_W1, m1_b1, m1_W2, m1_b2, m1_W3, m1_b3, m2_W1, m2_b1, m2_W2, m2_b2, m2_W3, m2_b3):
    raise NotImplementedError("write your pallas kernel here")


# speedup vs baseline: 76.7829x; 76.7829x over previous
import functools
import jax
import jax.numpy as jnp
from jax import lax
from jax.experimental import pallas as pl
from jax.experimental.pallas import tpu as pltpu
from jax.experimental.pallas import tpu_sc as plsc

NC = 2
NS = 16
NW = NC * NS
CH = 128
K = 8

_SC_PARAMS = pltpu.CompilerParams(use_tc_tiling_on_sc=False)


def _round_up(x, m):
    return (x + m - 1) // m * m


def _mesh():
    return plsc.VectorSubcoreMesh(core_axis_name="c", subcore_axis_name="s",
                                  num_cores=NC, num_subcores=NS)


def _make_edge_pass1(n_pad, e_pad, interpret=False):
    ept = e_pad // NW
    nb = ept // (CH * K)
    rpt = n_pad // NS

    @functools.partial(
        pl.kernel,
        out_type=jax.ShapeDtypeStruct((NC, n_pad), jnp.float32),
        mesh=_mesh(),
        scratch_types=[
            pltpu.VMEM((K, CH), jnp.int32),
            pltpu.VMEM((K, CH), jnp.int32),
            pltpu.VMEM((K, CH), jnp.float32),
            pltpu.VMEM_SHARED((n_pad,), jnp.float32),
            pltpu.VMEM_SHARED((n_pad,), jnp.float32),
            pltpu.SemaphoreType.DMA,
            pltpu.SemaphoreType.DMA,
        ],
        compiler_params=_SC_PARAMS,
        interpret=interpret,
    )
    def kern(src_hbm, dst_hbm, table_hbm, zeros_hbm, out_hbm,
             sidx, didx, rows, table_s, agg_s, gsem, ssem):
        cid = lax.axis_index("c")
        sid = lax.axis_index("s")
        wid = sid * NC + cid
        r0 = sid * rpt
        pltpu.sync_copy(zeros_hbm.at[pl.ds(r0, rpt)], agg_s.at[pl.ds(r0, rpt)])
        pltpu.sync_copy(table_hbm.at[pl.ds(r0, rpt)], table_s.at[pl.ds(r0, rpt)])
        plsc.subcore_barrier()

        row_base = wid * (ept // CH)

        def body(b, carry):
            ro = row_base + b * K
            pltpu.sync_copy(src_hbm.at[pl.ds(ro, K)], sidx)
            pltpu.sync_copy(dst_hbm.at[pl.ds(ro, K)], didx)
            gh = [pltpu.async_copy(table_s.at[sidx.at[j]], rows.at[j], gsem)
                  for j in range(K)]
            for h in gh:
                h.wait()
            sh = [pltpu.async_copy(rows.at[j], agg_s.at[didx.at[j]], ssem,
                                   add=True)
                  for j in range(K)]
            for h in sh:
                h.wait()
            return carry

        lax.fori_loop(0, nb, body, 0)
        plsc.subcore_barrier()
        pltpu.sync_copy(agg_s.at[pl.ds(r0, rpt)],
                        out_hbm.at[cid, pl.ds(r0, rpt)])

    return kern


def _make_edge_pass4(n_pad, e_pad, interpret=False):
    K4 = 4
    ept = e_pad // NW
    nb = ept // (CH * K4)
    rpt = n_pad // NS

    @functools.partial(
        pl.kernel,
        out_type=jax.ShapeDtypeStruct((NC, 4, n_pad), jnp.float32),
        mesh=_mesh(),
        scratch_types=[
            pltpu.VMEM((K4, CH), jnp.int32),
            pltpu.VMEM((K4, CH), jnp.int32),
            pltpu.VMEM((4, K4, CH), jnp.float32),
            pltpu.VMEM_SHARED((4, n_pad), jnp.float32),
            pltpu.VMEM_SHARED((4, n_pad), jnp.float32),
            pltpu.SemaphoreType.DMA,
            pltpu.SemaphoreType.DMA,
        ],
        compiler_params=_SC_PARAMS,
        interpret=interpret,
    )
    def kern(src_hbm, dst_hbm, t0, t1, t2, t3, zeros_hbm, out_hbm,
             sidx, didx, rows, table_s, agg_s, gsem, ssem):
        cid = lax.axis_index("c")
        sid = lax.axis_index("s")
        wid = sid * NC + cid
        r0 = sid * rpt
        for p in range(4):
            pltpu.sync_copy(zeros_hbm.at[pl.ds(r0, rpt)],
                            agg_s.at[p, pl.ds(r0, rpt)])
        for p, t in enumerate([t0, t1, t2, t3]):
            pltpu.sync_copy(t.at[pl.ds(r0, rpt)], table_s.at[p, pl.ds(r0, rpt)])
        plsc.subcore_barrier()

        row_base = wid * (ept // CH)

        def body(b, carry):
            ro = row_base + b * K4
            pltpu.sync_copy(src_hbm.at[pl.ds(ro, K4)], sidx)
            pltpu.sync_copy(dst_hbm.at[pl.ds(ro, K4)], didx)
            gh = [pltpu.async_copy(table_s.at[p].at[sidx.at[j]],
                                   rows.at[p, j], gsem)
                  for j in range(K4) for p in range(4)]
            for h in gh:
                h.wait()
            sh = [pltpu.async_copy(rows.at[p, j], agg_s.at[p].at[didx.at[j]],
                                   ssem, add=True)
                  for j in range(K4) for p in range(4)]
            for h in sh:
                h.wait()
            return carry

        lax.fori_loop(0, nb, body, 0)
        plsc.subcore_barrier()
        for p in range(4):
            pltpu.sync_copy(agg_s.at[p, pl.ds(r0, rpt)],
                            out_hbm.at[cid, p, pl.ds(r0, rpt)])

    return kern


def _make_deg_pass(n_pad, e_pad, interpret=False):
    ept = e_pad // NW
    nb = ept // (CH * K)
    rpt = n_pad // NS

    @functools.partial(
        pl.kernel,
        out_type=jax.ShapeDtypeStruct((NC, n_pad), jnp.float32),
        mesh=_mesh(),
        scratch_types=[
            pltpu.VMEM((K, CH), jnp.int32),
            pltpu.VMEM((K, CH), jnp.float32),
            pltpu.VMEM_SHARED((n_pad,), jnp.float32),
            pltpu.SemaphoreType.DMA,
        ],
        compiler_params=_SC_PARAMS,
        interpret=interpret,
    )
    def kern(dst_hbm, ones_hbm, zeros_hbm, out_hbm, didx, rows, agg_s, ssem):
        cid = lax.axis_index("c")
        sid = lax.axis_index("s")
        wid = sid * NC + cid
        r0 = sid * rpt
        pltpu.sync_copy(zeros_hbm.at[pl.ds(r0, rpt)], agg_s.at[pl.ds(r0, rpt)])
        pltpu.sync_copy(ones_hbm, rows)
        plsc.subcore_barrier()

        row_base = wid * (ept // CH)

        def body(b, carry):
            ro = row_base + b * K
            pltpu.sync_copy(dst_hbm.at[pl.ds(ro, K)], didx)
            sh = [pltpu.async_copy(rows.at[j], agg_s.at[didx.at[j]], ssem,
                                   add=True)
                  for j in range(K)]
            for h in sh:
                h.wait()
            return carry

        lax.fori_loop(0, nb, body, 0)
        plsc.subcore_barrier()
        pltpu.sync_copy(agg_s.at[pl.ds(r0, rpt)],
                        out_hbm.at[cid, pl.ds(r0, rpt)])

    return kern


def _tc_call(fn, ins, interpret=False):
    def body(*refs):
        in_refs = refs[:len(ins)]
        out_refs = refs[len(ins):]
        res = fn(*[r[...] for r in in_refs])
        for o_ref, o in zip(out_refs, res):
            o_ref[...] = o

    shapes = jax.eval_shape(lambda *a: fn(*a), *ins)
    return pl.pallas_call(
        body,
        out_shape=[jax.ShapeDtypeStruct(s.shape, s.dtype) for s in shapes],
        interpret=interpret,
    )(*ins)


def _bf(v):
    return v.astype(jnp.bfloat16).astype(jnp.float32)


def _plane_matmul(planes, W):
    din, dout = W.shape
    out = []
    for j in range(dout):
        acc = planes[0] * W[0, j]
        for i in range(1, din):
            acc = acc + planes[i] * W[i, j]
        out.append(acc)
    return out


def kernel(x, edge_index, m1_W1, m1_b1, m1_W2, m1_b2, m1_W3, m1_b3,
           m2_W1, m2_b1, m2_W2, m2_b2, m2_W3, m2_b3, interpret=False):
    n = x.shape[0]
    e = edge_index.shape[1]
    n_pad = _round_up(n + 1, 128)
    e_pad = _round_up(e, NW * CH * K)

    src = edge_index[0]
    dst = edge_index[1]
    pad_e = e_pad - e
    dummy = jnp.full((pad_e,), n, dtype=jnp.int32)
    src_r = jnp.concatenate([src, dummy]).reshape(e_pad // CH, CH)
    dst_r = jnp.concatenate([dst, dummy]).reshape(e_pad // CH, CH)

    rows128 = n_pad // 128
    x_plane = jnp.pad(x[:, 0], (0, n_pad - n)).reshape(rows128, 128)

    z1 = jnp.zeros((n_pad,), jnp.float32)
    ones = jnp.ones((K, CH), jnp.float32)

    b1_1 = m1_b1.reshape(1, 4)
    b2_1 = m1_b2.reshape(1, 4)
    b3_1 = m1_b3.reshape(1, 1)
    b1_2 = m2_b1.reshape(1, 4)
    b2_2 = m2_b2.reshape(1, 4)
    b3_2 = m2_b3.reshape(1, 1)

    deg_pass = _make_deg_pass(n_pad, e_pad, interpret)
    pass1 = _make_edge_pass1(n_pad, e_pad, interpret)
    pass4 = _make_edge_pass4(n_pad, e_pad, interpret)

    def plane(v):
        return v.reshape(rows128, 128)

    dcnt = deg_pass(dst_r, ones, z1)

    def e0(d0, d1, xp):
        deg = d0 + d1 + 1.0
        dinv = lax.rsqrt(deg)
        t1 = 10.0 * xp * dinv
        return (dinv, t1)
    dinv_p, t1_p = _tc_call(e0, [plane(dcnt[0]), plane(dcnt[1]), x_plane],
                            interpret=interpret)

    def gcn_stage(t1_p, W1, b1, W2, b2, W3, b3):
        s = pass1(src_r, dst_r, t1_p.reshape(n_pad), z1)

        def e1(s0, s1, t1, dinv, W1_, b1_, W2_):
            u1 = dinv * (s0 + s1 + t1)
            o1 = [jnp.maximum(u1 * W1_[0, j] + b1_[0, j], 0.0)
                  for j in range(4)]
            h2 = _plane_matmul([_bf(o) for o in o1], _bf(W2_))
            g2 = [h * dinv for h in h2]
            return tuple(g2)
        g2 = _tc_call(e1, [plane(s[0]), plane(s[1]), t1_p, dinv_p, W1, b1, W2],
                      interpret=interpret)

        a = pass4(src_r, dst_r, g2[0].reshape(n_pad), g2[1].reshape(n_pad),
                  g2[2].reshape(n_pad), g2[3].reshape(n_pad), z1)

        def e2(a0_0, a0_1, a0_2, a0_3, a1_0, a1_1, a1_2, a1_3,
               g0, g1, g2c, g3, dinv, b2_, W3_):
            ag0 = [a0_0, a0_1, a0_2, a0_3]
            ag1 = [a1_0, a1_1, a1_2, a1_3]
            g = [g0, g1, g2c, g3]
            o2 = [jnp.maximum(dinv * (ag0[j] + ag1[j] + g[j]) + b2_[0, j], 0.0)
                  for j in range(4)]
            W3b = _bf(W3_)
            h3 = _bf(o2[0]) * W3b[0, 0]
            for i in range(1, 4):
                h3 = h3 + _bf(o2[i]) * W3b[i, 0]
            t3 = h3 * dinv
            return (t3,)
        apl = [a[c, p].reshape(rows128, 128) for c in range(2)
               for p in range(4)]
        (t3_p,) = _tc_call(e2, apl + list(g2) +
                           [dinv_p, b2, W3], interpret=interpret)

        s3 = pass1(src_r, dst_r, t3_p.reshape(n_pad), z1)
        return s3, t3_p

    s3, t3_p = gcn_stage(t1_p, m1_W1, b1_1, m1_W2, b2_1, m1_W3, b3_1)

    def e3(s0, s1, t3, dinv, b3_):
        o3 = dinv * (s0 + s1 + t3) + b3_[0, 0]
        x2 = jnp.sin(o3 * 10.0)
        t1b = x2 * dinv
        return (x2, t1b)
    x2_p, t1b_p = _tc_call(e3, [plane(s3[0]), plane(s3[1]), t3_p, dinv_p,
                                b3_1], interpret=interpret)

    s3b, t3b_p = gcn_stage(t1b_p, m2_W1, b1_2, m2_W2, b2_2, m2_W3, b3_2)

    def e6(s0, s1, t3, dinv, b3_, x2):
        x3 = dinv * (s0 + s1 + t3) + b3_[0, 0]
        r = lax.broadcasted_iota(jnp.int32, x3.shape, 0)
        c = lax.broadcasted_iota(jnp.int32, x3.shape, 1)
        mask = (r * 128 + c) < n
        sx2 = jnp.sum(jnp.where(mask, x2, 0.0)).reshape(1, 1)
        sx3 = jnp.sum(jnp.where(mask, x3, 0.0)).reshape(1, 1)
        return (x3, sx2, sx3)
    x3_p, sx2, sx3 = _tc_call(e6, [plane(s3b[0]), plane(s3b[1]), t3b_p,
                                   dinv_p, b3_2, x2_p], interpret=interpret)

    x2_col = x2_p.reshape(n_pad)[:n]
    x3_col = x3_p.reshape(n_pad)[:n]
    node_emb = jnp.stack([x2_col, x3_col], axis=1)
    graph_emb = jnp.concatenate([sx2[0], sx3[0]]) / n
    return (graph_emb, node_emb)


# speedup vs baseline: 97.9745x; 1.2760x over previous
import functools
import jax
import jax.numpy as jnp
from jax import lax
from jax.experimental import pallas as pl
from jax.experimental.pallas import tpu as pltpu
from jax.experimental.pallas import tpu_sc as plsc

NC = 2
NS = 16
NW = NC * NS
CH = 128
K = 8

_SC_PARAMS = pltpu.CompilerParams(use_tc_tiling_on_sc=False)


def _round_up(x, m):
    return (x + m - 1) // m * m


def _mesh():
    return plsc.VectorSubcoreMesh(core_axis_name="c", subcore_axis_name="s",
                                  num_cores=NC, num_subcores=NS)


def _make_edge_pass1(n_pad, e_pad, interpret=False):
    ept = e_pad // NW
    nb = ept // (CH * K)
    rpt = n_pad // NS

    @functools.partial(
        pl.kernel,
        out_type=jax.ShapeDtypeStruct((NC, n_pad), jnp.float32),
        mesh=_mesh(),
        scratch_types=[
            pltpu.VMEM((2, K, CH), jnp.int32),
            pltpu.VMEM((2, K, CH), jnp.int32),
            pltpu.VMEM((K, CH), jnp.float32),
            pltpu.VMEM_SHARED((n_pad,), jnp.float32),
            pltpu.VMEM_SHARED((n_pad,), jnp.float32),
            pltpu.SemaphoreType.DMA,
            pltpu.SemaphoreType.DMA,
            pltpu.SemaphoreType.DMA,
        ],
        compiler_params=_SC_PARAMS,
        interpret=interpret,
    )
    def kern(src_hbm, dst_hbm, table_hbm, zeros_hbm, out_hbm,
             sidx, didx, rows, table_s, agg_s, isem, gsem, ssem):
        cid = lax.axis_index("c")
        sid = lax.axis_index("s")
        wid = sid * NC + cid
        r0 = sid * rpt
        pltpu.sync_copy(zeros_hbm.at[pl.ds(r0, rpt)], agg_s.at[pl.ds(r0, rpt)])
        pltpu.sync_copy(table_hbm.at[pl.ds(r0, rpt)], table_s.at[pl.ds(r0, rpt)])
        plsc.subcore_barrier()

        row_base = wid * (ept // CH)

        def fire_idx(ro, s):
            pltpu.async_copy(src_hbm.at[pl.ds(ro, K)], sidx.at[s], isem)
            pltpu.async_copy(dst_hbm.at[pl.ds(ro, K)], didx.at[s], isem)

        def drain_idx(s):
            pltpu.make_async_copy(src_hbm.at[pl.ds(0, K)], sidx.at[s],
                                  isem).wait()
            pltpu.make_async_copy(dst_hbm.at[pl.ds(0, K)], didx.at[s],
                                  isem).wait()

        fire_idx(row_base, 0)

        def body(c, carry):
            s = c % 2
            drain_idx(s)
            ro_next = jnp.where(c + 1 < nb, row_base + (c + 1) * K, row_base)
            fire_idx(ro_next, (c + 1) % 2)
            gh = [pltpu.async_copy(table_s.at[sidx.at[s, j]], rows.at[j], gsem)
                  for j in range(K)]
            for h in gh:
                h.wait()
            sh = [pltpu.async_copy(rows.at[j], agg_s.at[didx.at[s, j]], ssem,
                                   add=True)
                  for j in range(K)]
            for h in sh:
                h.wait()
            return carry

        lax.fori_loop(0, nb, body, 0)
        drain_idx(nb % 2)
        plsc.subcore_barrier()
        pltpu.sync_copy(agg_s.at[pl.ds(r0, rpt)],
                        out_hbm.at[cid, pl.ds(r0, rpt)])

    return kern


def _make_edge_pass4(n_pad, e_pad, interpret=False):
    K4 = 2
    ept = e_pad // NW
    nb = ept // (CH * K4)
    rpt = n_pad // NS

    @functools.partial(
        pl.kernel,
        out_type=jax.ShapeDtypeStruct((NC, 4, n_pad), jnp.float32),
        mesh=_mesh(),
        scratch_types=[
            pltpu.VMEM((2, K4, CH), jnp.int32),
            pltpu.VMEM((2, K4, CH), jnp.int32),
            pltpu.VMEM((4, K4, CH), jnp.float32),
            pltpu.VMEM_SHARED((4, n_pad), jnp.float32),
            pltpu.VMEM_SHARED((4, n_pad), jnp.float32),
            pltpu.SemaphoreType.DMA,
            pltpu.SemaphoreType.DMA,
            pltpu.SemaphoreType.DMA,
        ],
        compiler_params=_SC_PARAMS,
        interpret=interpret,
    )
    def kern(src_hbm, dst_hbm, t0, t1, t2, t3, zeros_hbm, out_hbm,
             sidx, didx, rows, table_s, agg_s, isem, gsem, ssem):
        cid = lax.axis_index("c")
        sid = lax.axis_index("s")
        wid = sid * NC + cid
        r0 = sid * rpt
        for p in range(4):
            pltpu.sync_copy(zeros_hbm.at[pl.ds(r0, rpt)],
                            agg_s.at[p, pl.ds(r0, rpt)])
        for p, t in enumerate([t0, t1, t2, t3]):
            pltpu.sync_copy(t.at[pl.ds(r0, rpt)], table_s.at[p, pl.ds(r0, rpt)])
        plsc.subcore_barrier()

        row_base = wid * (ept // CH)

        def fire_idx(ro, s):
            pltpu.async_copy(src_hbm.at[pl.ds(ro, K4)], sidx.at[s], isem)
            pltpu.async_copy(dst_hbm.at[pl.ds(ro, K4)], didx.at[s], isem)

        def drain_idx(s):
            pltpu.make_async_copy(src_hbm.at[pl.ds(0, K4)], sidx.at[s],
                                  isem).wait()
            pltpu.make_async_copy(dst_hbm.at[pl.ds(0, K4)], didx.at[s],
                                  isem).wait()

        fire_idx(row_base, 0)

        def body(c, carry):
            s = c % 2
            drain_idx(s)
            ro_next = jnp.where(c + 1 < nb, row_base + (c + 1) * K4, row_base)
            fire_idx(ro_next, (c + 1) % 2)
            gh = [pltpu.async_copy(table_s.at[p].at[sidx.at[s, j]],
                                   rows.at[p, j], gsem)
                  for j in range(K4) for p in range(4)]
            for h in gh:
                h.wait()
            sh = [pltpu.async_copy(rows.at[p, j], agg_s.at[p].at[didx.at[s, j]],
                                   ssem, add=True)
                  for j in range(K4) for p in range(4)]
            for h in sh:
                h.wait()
            return carry

        lax.fori_loop(0, nb, body, 0)
        drain_idx(nb % 2)
        plsc.subcore_barrier()
        for p in range(4):
            pltpu.sync_copy(agg_s.at[p, pl.ds(r0, rpt)],
                            out_hbm.at[cid, p, pl.ds(r0, rpt)])

    return kern


def _make_deg_pass(n_pad, e_pad, interpret=False):
    ept = e_pad // NW
    nb = ept // (CH * K)
    rpt = n_pad // NS

    @functools.partial(
        pl.kernel,
        out_type=jax.ShapeDtypeStruct((NC, n_pad), jnp.float32),
        mesh=_mesh(),
        scratch_types=[
            pltpu.VMEM((2, K, CH), jnp.int32),
            pltpu.VMEM((K, CH), jnp.float32),
            pltpu.VMEM_SHARED((n_pad,), jnp.float32),
            pltpu.SemaphoreType.DMA,
            pltpu.SemaphoreType.DMA,
        ],
        compiler_params=_SC_PARAMS,
        interpret=interpret,
    )
    def kern(dst_hbm, ones_hbm, zeros_hbm, out_hbm, didx, rows, agg_s,
             isem, ssem):
        cid = lax.axis_index("c")
        sid = lax.axis_index("s")
        wid = sid * NC + cid
        r0 = sid * rpt
        pltpu.sync_copy(zeros_hbm.at[pl.ds(r0, rpt)], agg_s.at[pl.ds(r0, rpt)])
        pltpu.sync_copy(ones_hbm, rows)
        plsc.subcore_barrier()

        row_base = wid * (ept // CH)

        def fire_idx(ro, s):
            pltpu.async_copy(dst_hbm.at[pl.ds(ro, K)], didx.at[s], isem)

        def drain_idx(s):
            pltpu.make_async_copy(dst_hbm.at[pl.ds(0, K)], didx.at[s],
                                  isem).wait()

        fire_idx(row_base, 0)

        def body(c, carry):
            s = c % 2
            drain_idx(s)
            ro_next = jnp.where(c + 1 < nb, row_base + (c + 1) * K, row_base)
            fire_idx(ro_next, (c + 1) % 2)
            sh = [pltpu.async_copy(rows.at[j], agg_s.at[didx.at[s, j]], ssem,
                                   add=True)
                  for j in range(K)]
            for h in sh:
                h.wait()
            return carry

        lax.fori_loop(0, nb, body, 0)
        drain_idx(nb % 2)
        plsc.subcore_barrier()
        pltpu.sync_copy(agg_s.at[pl.ds(r0, rpt)],
                        out_hbm.at[cid, pl.ds(r0, rpt)])

    return kern


def _tc_call(fn, ins, interpret=False):
    def body(*refs):
        in_refs = refs[:len(ins)]
        out_refs = refs[len(ins):]
        res = fn(*[r[...] for r in in_refs])
        for o_ref, o in zip(out_refs, res):
            o_ref[...] = o

    shapes = jax.eval_shape(lambda *a: fn(*a), *ins)
    return pl.pallas_call(
        body,
        out_shape=[jax.ShapeDtypeStruct(s.shape, s.dtype) for s in shapes],
        interpret=interpret,
    )(*ins)


def _bf(v):
    return v.astype(jnp.bfloat16).astype(jnp.float32)


def _plane_matmul(planes, W):
    din, dout = W.shape
    out = []
    for j in range(dout):
        acc = planes[0] * W[0, j]
        for i in range(1, din):
            acc = acc + planes[i] * W[i, j]
        out.append(acc)
    return out


def kernel(x, edge_index, m1_W1, m1_b1, m1_W2, m1_b2, m1_W3, m1_b3,
           m2_W1, m2_b1, m2_W2, m2_b2, m2_W3, m2_b3, interpret=False):
    n = x.shape[0]
    e = edge_index.shape[1]
    n_pad = _round_up(n + 1, 128)
    e_pad = _round_up(e, NW * CH * K)

    src = edge_index[0]
    dst = edge_index[1]
    pad_e = e_pad - e
    dummy = jnp.full((pad_e,), n, dtype=jnp.int32)
    src_r = jnp.concatenate([src, dummy]).reshape(e_pad // CH, CH)
    dst_r = jnp.concatenate([dst, dummy]).reshape(e_pad // CH, CH)

    rows128 = n_pad // 128
    x_plane = jnp.pad(x[:, 0], (0, n_pad - n)).reshape(rows128, 128)

    z1 = jnp.zeros((n_pad,), jnp.float32)
    ones = jnp.ones((K, CH), jnp.float32)

    b1_1 = m1_b1.reshape(1, 4)
    b2_1 = m1_b2.reshape(1, 4)
    b3_1 = m1_b3.reshape(1, 1)
    b1_2 = m2_b1.reshape(1, 4)
    b2_2 = m2_b2.reshape(1, 4)
    b3_2 = m2_b3.reshape(1, 1)

    deg_pass = _make_deg_pass(n_pad, e_pad, interpret)
    pass1 = _make_edge_pass1(n_pad, e_pad, interpret)
    pass4 = _make_edge_pass4(n_pad, e_pad, interpret)

    def plane(v):
        return v.reshape(rows128, 128)

    dcnt = deg_pass(dst_r, ones, z1)

    def e0(d0, d1, xp):
        deg = d0 + d1 + 1.0
        dinv = lax.rsqrt(deg)
        t1 = 10.0 * xp * dinv
        return (dinv, t1)
    dinv_p, t1_p = _tc_call(e0, [plane(dcnt[0]), plane(dcnt[1]), x_plane],
                            interpret=interpret)

    def gcn_stage(t1_p, W1, b1, W2, b2, W3, b3):
        s = pass1(src_r, dst_r, t1_p.reshape(n_pad), z1)

        def e1(s0, s1, t1, dinv, W1_, b1_, W2_):
            u1 = dinv * (s0 + s1 + t1)
            o1 = [jnp.maximum(u1 * W1_[0, j] + b1_[0, j], 0.0)
                  for j in range(4)]
            h2 = _plane_matmul([_bf(o) for o in o1], _bf(W2_))
            g2 = [h * dinv for h in h2]
            return tuple(g2)
        g2 = _tc_call(e1, [plane(s[0]), plane(s[1]), t1_p, dinv_p, W1, b1, W2],
                      interpret=interpret)

        a = pass4(src_r, dst_r, g2[0].reshape(n_pad), g2[1].reshape(n_pad),
                  g2[2].reshape(n_pad), g2[3].reshape(n_pad), z1)

        def e2(a0_0, a0_1, a0_2, a0_3, a1_0, a1_1, a1_2, a1_3,
               g0, g1, g2c, g3, dinv, b2_, W3_):
            ag0 = [a0_0, a0_1, a0_2, a0_3]
            ag1 = [a1_0, a1_1, a1_2, a1_3]
            g = [g0, g1, g2c, g3]
            o2 = [jnp.maximum(dinv * (ag0[j] + ag1[j] + g[j]) + b2_[0, j], 0.0)
                  for j in range(4)]
            W3b = _bf(W3_)
            h3 = _bf(o2[0]) * W3b[0, 0]
            for i in range(1, 4):
                h3 = h3 + _bf(o2[i]) * W3b[i, 0]
            t3 = h3 * dinv
            return (t3,)
        apl = [a[c, p].reshape(rows128, 128) for c in range(2)
               for p in range(4)]
        (t3_p,) = _tc_call(e2, apl + list(g2) +
                           [dinv_p, b2, W3], interpret=interpret)

        s3 = pass1(src_r, dst_r, t3_p.reshape(n_pad), z1)
        return s3, t3_p

    s3, t3_p = gcn_stage(t1_p, m1_W1, b1_1, m1_W2, b2_1, m1_W3, b3_1)

    def e3(s0, s1, t3, dinv, b3_):
        o3 = dinv * (s0 + s1 + t3) + b3_[0, 0]
        x2 = jnp.sin(o3 * 10.0)
        t1b = x2 * dinv
        return (x2, t1b)
    x2_p, t1b_p = _tc_call(e3, [plane(s3[0]), plane(s3[1]), t3_p, dinv_p,
                                b3_1], interpret=interpret)

    s3b, t3b_p = gcn_stage(t1b_p, m2_W1, b1_2, m2_W2, b2_2, m2_W3, b3_2)

    def e6(s0, s1, t3, dinv, b3_, x2):
        x3 = dinv * (s0 + s1 + t3) + b3_[0, 0]
        r = lax.broadcasted_iota(jnp.int32, x3.shape, 0)
        c = lax.broadcasted_iota(jnp.int32, x3.shape, 1)
        mask = (r * 128 + c) < n
        sx2 = jnp.sum(jnp.where(mask, x2, 0.0)).reshape(1, 1)
        sx3 = jnp.sum(jnp.where(mask, x3, 0.0)).reshape(1, 1)
        return (x3, sx2, sx3)
    x3_p, sx2, sx3 = _tc_call(e6, [plane(s3b[0]), plane(s3b[1]), t3b_p,
                                   dinv_p, b3_2, x2_p], interpret=interpret)

    x2_col = x2_p.reshape(n_pad)[:n]
    x3_col = x3_p.reshape(n_pad)[:n]
    node_emb = jnp.stack([x2_col, x3_col], axis=1)
    graph_emb = jnp.concatenate([sx2[0], sx3[0]]) / n
    return (graph_emb, node_emb)


# speedup vs baseline: 99.1787x; 1.0123x over previous
import functools
import jax
import jax.numpy as jnp
from jax import lax
from jax.experimental import pallas as pl
from jax.experimental.pallas import tpu as pltpu
from jax.experimental.pallas import tpu_sc as plsc

NC = 2
NS = 16
NW = NC * NS
CH = 128
K = 16

_SC_PARAMS = pltpu.CompilerParams(use_tc_tiling_on_sc=False)


def _round_up(x, m):
    return (x + m - 1) // m * m


def _mesh():
    return plsc.VectorSubcoreMesh(core_axis_name="c", subcore_axis_name="s",
                                  num_cores=NC, num_subcores=NS)


def _make_edge_pass1(n_pad, e_pad, interpret=False):
    ept = e_pad // NW
    nb = ept // (CH * K)
    rpt = n_pad // NS

    @functools.partial(
        pl.kernel,
        out_type=jax.ShapeDtypeStruct((NC, n_pad), jnp.float32),
        mesh=_mesh(),
        scratch_types=[
            pltpu.VMEM((2, K, CH), jnp.int32),
            pltpu.VMEM((2, K, CH), jnp.int32),
            pltpu.VMEM((K, CH), jnp.float32),
            pltpu.VMEM_SHARED((n_pad,), jnp.float32),
            pltpu.VMEM_SHARED((n_pad,), jnp.float32),
            pltpu.SemaphoreType.DMA,
            pltpu.SemaphoreType.DMA,
            pltpu.SemaphoreType.DMA,
        ],
        compiler_params=_SC_PARAMS,
        interpret=interpret,
    )
    def kern(src_hbm, dst_hbm, table_hbm, zeros_hbm, out_hbm,
             sidx, didx, rows, table_s, agg_s, isem, gsem, ssem):
        cid = lax.axis_index("c")
        sid = lax.axis_index("s")
        wid = sid * NC + cid
        r0 = sid * rpt
        pltpu.sync_copy(zeros_hbm.at[pl.ds(r0, rpt)], agg_s.at[pl.ds(r0, rpt)])
        pltpu.sync_copy(table_hbm.at[pl.ds(r0, rpt)], table_s.at[pl.ds(r0, rpt)])
        plsc.subcore_barrier()

        row_base = wid * (ept // CH)

        def fire_idx(ro, s):
            pltpu.async_copy(src_hbm.at[pl.ds(ro, K)], sidx.at[s], isem)
            pltpu.async_copy(dst_hbm.at[pl.ds(ro, K)], didx.at[s], isem)

        def drain_idx(s):
            pltpu.make_async_copy(src_hbm.at[pl.ds(0, K)], sidx.at[s],
                                  isem).wait()
            pltpu.make_async_copy(dst_hbm.at[pl.ds(0, K)], didx.at[s],
                                  isem).wait()

        fire_idx(row_base, 0)

        def body(c, carry):
            s = c % 2
            drain_idx(s)
            ro_next = jnp.where(c + 1 < nb, row_base + (c + 1) * K, row_base)
            fire_idx(ro_next, (c + 1) % 2)
            gh = [pltpu.async_copy(table_s.at[sidx.at[s, j]], rows.at[j], gsem)
                  for j in range(K)]
            for h in gh:
                h.wait()
            sh = [pltpu.async_copy(rows.at[j], agg_s.at[didx.at[s, j]], ssem,
                                   add=True)
                  for j in range(K)]
            for h in sh:
                h.wait()
            return carry

        lax.fori_loop(0, nb, body, 0)
        drain_idx(nb % 2)
        plsc.subcore_barrier()
        pltpu.sync_copy(agg_s.at[pl.ds(r0, rpt)],
                        out_hbm.at[cid, pl.ds(r0, rpt)])

    return kern


def _make_edge_pass4(n_pad, e_pad, interpret=False):
    K4 = 2
    ept = e_pad // NW
    nb = ept // (CH * K4)
    rpt = n_pad // NS

    @functools.partial(
        pl.kernel,
        out_type=jax.ShapeDtypeStruct((NC, 4, n_pad), jnp.float32),
        mesh=_mesh(),
        scratch_types=[
            pltpu.VMEM((2, K4, CH), jnp.int32),
            pltpu.VMEM((2, K4, CH), jnp.int32),
            pltpu.VMEM((4, K4, CH), jnp.float32),
            pltpu.VMEM_SHARED((4, n_pad), jnp.float32),
            pltpu.VMEM_SHARED((4, n_pad), jnp.float32),
            pltpu.SemaphoreType.DMA,
            pltpu.SemaphoreType.DMA,
            pltpu.SemaphoreType.DMA,
        ],
        compiler_params=_SC_PARAMS,
        interpret=interpret,
    )
    def kern(src_hbm, dst_hbm, t0, t1, t2, t3, zeros_hbm, out_hbm,
             sidx, didx, rows, table_s, agg_s, isem, gsem, ssem):
        cid = lax.axis_index("c")
        sid = lax.axis_index("s")
        wid = sid * NC + cid
        r0 = sid * rpt
        for p in range(4):
            pltpu.sync_copy(zeros_hbm.at[pl.ds(r0, rpt)],
                            agg_s.at[p, pl.ds(r0, rpt)])
        for p, t in enumerate([t0, t1, t2, t3]):
            pltpu.sync_copy(t.at[pl.ds(r0, rpt)], table_s.at[p, pl.ds(r0, rpt)])
        plsc.subcore_barrier()

        row_base = wid * (ept // CH)

        def fire_idx(ro, s):
            pltpu.async_copy(src_hbm.at[pl.ds(ro, K4)], sidx.at[s], isem)
            pltpu.async_copy(dst_hbm.at[pl.ds(ro, K4)], didx.at[s], isem)

        def drain_idx(s):
            pltpu.make_async_copy(src_hbm.at[pl.ds(0, K4)], sidx.at[s],
                                  isem).wait()
            pltpu.make_async_copy(dst_hbm.at[pl.ds(0, K4)], didx.at[s],
                                  isem).wait()

        fire_idx(row_base, 0)

        def body(c, carry):
            s = c % 2
            drain_idx(s)
            ro_next = jnp.where(c + 1 < nb, row_base + (c + 1) * K4, row_base)
            fire_idx(ro_next, (c + 1) % 2)
            gh = [pltpu.async_copy(table_s.at[p].at[sidx.at[s, j]],
                                   rows.at[p, j], gsem)
                  for j in range(K4) for p in range(4)]
            for h in gh:
                h.wait()
            sh = [pltpu.async_copy(rows.at[p, j], agg_s.at[p].at[didx.at[s, j]],
                                   ssem, add=True)
                  for j in range(K4) for p in range(4)]
            for h in sh:
                h.wait()
            return carry

        lax.fori_loop(0, nb, body, 0)
        drain_idx(nb % 2)
        plsc.subcore_barrier()
        for p in range(4):
            pltpu.sync_copy(agg_s.at[p, pl.ds(r0, rpt)],
                            out_hbm.at[cid, p, pl.ds(r0, rpt)])

    return kern


def _make_deg_pass(n_pad, e_pad, interpret=False):
    ept = e_pad // NW
    nb = ept // (CH * K)
    rpt = n_pad // NS

    @functools.partial(
        pl.kernel,
        out_type=jax.ShapeDtypeStruct((NC, n_pad), jnp.float32),
        mesh=_mesh(),
        scratch_types=[
            pltpu.VMEM((2, K, CH), jnp.int32),
            pltpu.VMEM((K, CH), jnp.float32),
            pltpu.VMEM_SHARED((n_pad,), jnp.float32),
            pltpu.SemaphoreType.DMA,
            pltpu.SemaphoreType.DMA,
        ],
        compiler_params=_SC_PARAMS,
        interpret=interpret,
    )
    def kern(dst_hbm, ones_hbm, zeros_hbm, out_hbm, didx, rows, agg_s,
             isem, ssem):
        cid = lax.axis_index("c")
        sid = lax.axis_index("s")
        wid = sid * NC + cid
        r0 = sid * rpt
        pltpu.sync_copy(zeros_hbm.at[pl.ds(r0, rpt)], agg_s.at[pl.ds(r0, rpt)])
        pltpu.sync_copy(ones_hbm, rows)
        plsc.subcore_barrier()

        row_base = wid * (ept // CH)

        def fire_idx(ro, s):
            pltpu.async_copy(dst_hbm.at[pl.ds(ro, K)], didx.at[s], isem)

        def drain_idx(s):
            pltpu.make_async_copy(dst_hbm.at[pl.ds(0, K)], didx.at[s],
                                  isem).wait()

        fire_idx(row_base, 0)

        def body(c, carry):
            s = c % 2
            drain_idx(s)
            ro_next = jnp.where(c + 1 < nb, row_base + (c + 1) * K, row_base)
            fire_idx(ro_next, (c + 1) % 2)
            sh = [pltpu.async_copy(rows.at[j], agg_s.at[didx.at[s, j]], ssem,
                                   add=True)
                  for j in range(K)]
            for h in sh:
                h.wait()
            return carry

        lax.fori_loop(0, nb, body, 0)
        drain_idx(nb % 2)
        plsc.subcore_barrier()
        pltpu.sync_copy(agg_s.at[pl.ds(r0, rpt)],
                        out_hbm.at[cid, pl.ds(r0, rpt)])

    return kern


def _tc_call(fn, ins, interpret=False):
    def body(*refs):
        in_refs = refs[:len(ins)]
        out_refs = refs[len(ins):]
        res = fn(*[r[...] for r in in_refs])
        for o_ref, o in zip(out_refs, res):
            o_ref[...] = o

    shapes = jax.eval_shape(lambda *a: fn(*a), *ins)
    return pl.pallas_call(
        body,
        out_shape=[jax.ShapeDtypeStruct(s.shape, s.dtype) for s in shapes],
        interpret=interpret,
    )(*ins)


def _bf(v):
    return v.astype(jnp.bfloat16).astype(jnp.float32)


def _plane_matmul(planes, W):
    din, dout = W.shape
    out = []
    for j in range(dout):
        acc = planes[0] * W[0, j]
        for i in range(1, din):
            acc = acc + planes[i] * W[i, j]
        out.append(acc)
    return out


def kernel(x, edge_index, m1_W1, m1_b1, m1_W2, m1_b2, m1_W3, m1_b3,
           m2_W1, m2_b1, m2_W2, m2_b2, m2_W3, m2_b3, interpret=False):
    n = x.shape[0]
    e = edge_index.shape[1]
    n_pad = _round_up(n + 1, 128)
    e_pad = _round_up(e, NW * CH * K)

    src = edge_index[0]
    dst = edge_index[1]
    pad_e = e_pad - e
    dummy = jnp.full((pad_e,), n, dtype=jnp.int32)
    src_r = jnp.concatenate([src, dummy]).reshape(e_pad // CH, CH)
    dst_r = jnp.concatenate([dst, dummy]).reshape(e_pad // CH, CH)

    rows128 = n_pad // 128
    x_plane = jnp.pad(x[:, 0], (0, n_pad - n)).reshape(rows128, 128)

    z1 = jnp.zeros((n_pad,), jnp.float32)
    ones = jnp.ones((K, CH), jnp.float32)

    b1_1 = m1_b1.reshape(1, 4)
    b2_1 = m1_b2.reshape(1, 4)
    b3_1 = m1_b3.reshape(1, 1)
    b1_2 = m2_b1.reshape(1, 4)
    b2_2 = m2_b2.reshape(1, 4)
    b3_2 = m2_b3.reshape(1, 1)

    deg_pass = _make_deg_pass(n_pad, e_pad, interpret)
    pass1 = _make_edge_pass1(n_pad, e_pad, interpret)
    pass4 = _make_edge_pass4(n_pad, e_pad, interpret)

    def plane(v):
        return v.reshape(rows128, 128)

    dcnt = deg_pass(dst_r, ones, z1)

    def e0(d0, d1, xp):
        deg = d0 + d1 + 1.0
        dinv = lax.rsqrt(deg)
        t1 = 10.0 * xp * dinv
        return (dinv, t1)
    dinv_p, t1_p = _tc_call(e0, [plane(dcnt[0]), plane(dcnt[1]), x_plane],
                            interpret=interpret)

    def gcn_stage(t1_p, W1, b1, W2, b2, W3, b3):
        s = pass1(src_r, dst_r, t1_p.reshape(n_pad), z1)

        def e1(s0, s1, t1, dinv, W1_, b1_, W2_):
            u1 = dinv * (s0 + s1 + t1)
            o1 = [jnp.maximum(u1 * W1_[0, j] + b1_[0, j], 0.0)
                  for j in range(4)]
            h2 = _plane_matmul([_bf(o) for o in o1], _bf(W2_))
            g2 = [h * dinv for h in h2]
            return tuple(g2)
        g2 = _tc_call(e1, [plane(s[0]), plane(s[1]), t1_p, dinv_p, W1, b1, W2],
                      interpret=interpret)

        a = pass4(src_r, dst_r, g2[0].reshape(n_pad), g2[1].reshape(n_pad),
                  g2[2].reshape(n_pad), g2[3].reshape(n_pad), z1)

        def e2(a0_0, a0_1, a0_2, a0_3, a1_0, a1_1, a1_2, a1_3,
               g0, g1, g2c, g3, dinv, b2_, W3_):
            ag0 = [a0_0, a0_1, a0_2, a0_3]
            ag1 = [a1_0, a1_1, a1_2, a1_3]
            g = [g0, g1, g2c, g3]
            o2 = [jnp.maximum(dinv * (ag0[j] + ag1[j] + g[j]) + b2_[0, j], 0.0)
                  for j in range(4)]
            W3b = _bf(W3_)
            h3 = _bf(o2[0]) * W3b[0, 0]
            for i in range(1, 4):
                h3 = h3 + _bf(o2[i]) * W3b[i, 0]
            t3 = h3 * dinv
            return (t3,)
        apl = [a[c, p].reshape(rows128, 128) for c in range(2)
               for p in range(4)]
        (t3_p,) = _tc_call(e2, apl + list(g2) +
                           [dinv_p, b2, W3], interpret=interpret)

        s3 = pass1(src_r, dst_r, t3_p.reshape(n_pad), z1)
        return s3, t3_p

    s3, t3_p = gcn_stage(t1_p, m1_W1, b1_1, m1_W2, b2_1, m1_W3, b3_1)

    def e3(s0, s1, t3, dinv, b3_):
        o3 = dinv * (s0 + s1 + t3) + b3_[0, 0]
        x2 = jnp.sin(o3 * 10.0)
        t1b = x2 * dinv
        return (x2, t1b)
    x2_p, t1b_p = _tc_call(e3, [plane(s3[0]), plane(s3[1]), t3_p, dinv_p,
                                b3_1], interpret=interpret)

    s3b, t3b_p = gcn_stage(t1b_p, m2_W1, b1_2, m2_W2, b2_2, m2_W3, b3_2)

    def e6(s0, s1, t3, dinv, b3_, x2):
        x3 = dinv * (s0 + s1 + t3) + b3_[0, 0]
        r = lax.broadcasted_iota(jnp.int32, x3.shape, 0)
        c = lax.broadcasted_iota(jnp.int32, x3.shape, 1)
        mask = (r * 128 + c) < n
        sx2 = jnp.sum(jnp.where(mask, x2, 0.0)).reshape(1, 1)
        sx3 = jnp.sum(jnp.where(mask, x3, 0.0)).reshape(1, 1)
        return (x3, sx2, sx3)
    x3_p, sx2, sx3 = _tc_call(e6, [plane(s3b[0]), plane(s3b[1]), t3b_p,
                                   dinv_p, b3_2, x2_p], interpret=interpret)

    x2_col = x2_p.reshape(n_pad)[:n]
    x3_col = x3_p.reshape(n_pad)[:n]
    node_emb = jnp.stack([x2_col, x3_col], axis=1)
    graph_emb = jnp.concatenate([sx2[0], sx3[0]]) / n
    return (graph_emb, node_emb)


# speedup vs baseline: 99.2198x; 1.0004x over previous
import functools
import jax
import jax.numpy as jnp
from jax import lax
from jax.experimental import pallas as pl
from jax.experimental.pallas import tpu as pltpu
from jax.experimental.pallas import tpu_sc as plsc

NC = 2
NS = 16
NW = NC * NS
CH = 128
K = 16

_SC_PARAMS = pltpu.CompilerParams(use_tc_tiling_on_sc=False)


def _round_up(x, m):
    return (x + m - 1) // m * m


def _mesh():
    return plsc.VectorSubcoreMesh(core_axis_name="c", subcore_axis_name="s",
                                  num_cores=NC, num_subcores=NS)


def _make_edge_pass1(n_pad, e_pad):
    ept = e_pad // NW
    nb = ept // (CH * K)
    rpt = n_pad // NS

    @functools.partial(
        pl.kernel,
        out_type=jax.ShapeDtypeStruct((NC, n_pad), jnp.float32),
        mesh=_mesh(),
        scratch_types=[
            pltpu.VMEM((2, K, CH), jnp.int32),
            pltpu.VMEM((2, K, CH), jnp.int32),
            pltpu.VMEM((K, CH), jnp.float32),
            pltpu.VMEM_SHARED((n_pad,), jnp.float32),
            pltpu.VMEM_SHARED((n_pad,), jnp.float32),
            pltpu.SemaphoreType.DMA,
            pltpu.SemaphoreType.DMA,
            pltpu.SemaphoreType.DMA,
        ],
        compiler_params=_SC_PARAMS,
    )
    def kern(src_hbm, dst_hbm, table_hbm, zeros_hbm, out_hbm,
             sidx, didx, rows, table_s, agg_s, isem, gsem, ssem):
        cid = lax.axis_index("c")
        sid = lax.axis_index("s")
        wid = sid * NC + cid
        r0 = sid * rpt
        pltpu.sync_copy(zeros_hbm.at[pl.ds(r0, rpt)], agg_s.at[pl.ds(r0, rpt)])
        pltpu.sync_copy(table_hbm.at[pl.ds(r0, rpt)], table_s.at[pl.ds(r0, rpt)])
        plsc.subcore_barrier()

        row_base = wid * (ept // CH)

        def fire_idx(ro, s):
            pltpu.async_copy(src_hbm.at[pl.ds(ro, K)], sidx.at[s], isem)
            pltpu.async_copy(dst_hbm.at[pl.ds(ro, K)], didx.at[s], isem)

        def drain_idx(s):
            pltpu.make_async_copy(src_hbm.at[pl.ds(0, K)], sidx.at[s],
                                  isem).wait()
            pltpu.make_async_copy(dst_hbm.at[pl.ds(0, K)], didx.at[s],
                                  isem).wait()

        fire_idx(row_base, 0)

        def body(c, carry):
            s = c % 2
            drain_idx(s)
            ro_next = jnp.where(c + 1 < nb, row_base + (c + 1) * K, row_base)
            fire_idx(ro_next, (c + 1) % 2)
            gh = [pltpu.async_copy(table_s.at[sidx.at[s, j]], rows.at[j], gsem)
                  for j in range(K)]
            for h in gh:
                h.wait()
            sh = [pltpu.async_copy(rows.at[j], agg_s.at[didx.at[s, j]], ssem,
                                   add=True)
                  for j in range(K)]
            for h in sh:
                h.wait()
            return carry

        lax.fori_loop(0, nb, body, 0)
        drain_idx(nb % 2)
        plsc.subcore_barrier()
        pltpu.sync_copy(agg_s.at[pl.ds(r0, rpt)],
                        out_hbm.at[cid, pl.ds(r0, rpt)])

    return kern


def _make_edge_pass4(n_pad, e_pad):
    K4 = 2
    ept = e_pad // NW
    nb = ept // (CH * K4)
    rpt = n_pad // NS

    @functools.partial(
        pl.kernel,
        out_type=jax.ShapeDtypeStruct((NC, 4, n_pad), jnp.float32),
        mesh=_mesh(),
        scratch_types=[
            pltpu.VMEM((2, K4, CH), jnp.int32),
            pltpu.VMEM((2, K4, CH), jnp.int32),
            pltpu.VMEM((4, K4, CH), jnp.float32),
            pltpu.VMEM_SHARED((4, n_pad), jnp.float32),
            pltpu.VMEM_SHARED((4, n_pad), jnp.float32),
            pltpu.SemaphoreType.DMA,
            pltpu.SemaphoreType.DMA,
            pltpu.SemaphoreType.DMA,
        ],
        compiler_params=_SC_PARAMS,
    )
    def kern(src_hbm, dst_hbm, t0, t1, t2, t3, zeros_hbm, out_hbm,
             sidx, didx, rows, table_s, agg_s, isem, gsem, ssem):
        cid = lax.axis_index("c")
        sid = lax.axis_index("s")
        wid = sid * NC + cid
        r0 = sid * rpt
        for p in range(4):
            pltpu.sync_copy(zeros_hbm.at[pl.ds(r0, rpt)],
                            agg_s.at[p, pl.ds(r0, rpt)])
        for p, t in enumerate([t0, t1, t2, t3]):
            pltpu.sync_copy(t.at[pl.ds(r0, rpt)], table_s.at[p, pl.ds(r0, rpt)])
        plsc.subcore_barrier()

        row_base = wid * (ept // CH)

        def fire_idx(ro, s):
            pltpu.async_copy(src_hbm.at[pl.ds(ro, K4)], sidx.at[s], isem)
            pltpu.async_copy(dst_hbm.at[pl.ds(ro, K4)], didx.at[s], isem)

        def drain_idx(s):
            pltpu.make_async_copy(src_hbm.at[pl.ds(0, K4)], sidx.at[s],
                                  isem).wait()
            pltpu.make_async_copy(dst_hbm.at[pl.ds(0, K4)], didx.at[s],
                                  isem).wait()

        fire_idx(row_base, 0)

        def body(c, carry):
            s = c % 2
            drain_idx(s)
            ro_next = jnp.where(c + 1 < nb, row_base + (c + 1) * K4, row_base)
            fire_idx(ro_next, (c + 1) % 2)
            gh = [pltpu.async_copy(table_s.at[p].at[sidx.at[s, j]],
                                   rows.at[p, j], gsem)
                  for j in range(K4) for p in range(4)]
            for h in gh:
                h.wait()
            sh = [pltpu.async_copy(rows.at[p, j], agg_s.at[p].at[didx.at[s, j]],
                                   ssem, add=True)
                  for j in range(K4) for p in range(4)]
            for h in sh:
                h.wait()
            return carry

        lax.fori_loop(0, nb, body, 0)
        drain_idx(nb % 2)
        plsc.subcore_barrier()
        for p in range(4):
            pltpu.sync_copy(agg_s.at[p, pl.ds(r0, rpt)],
                            out_hbm.at[cid, p, pl.ds(r0, rpt)])

    return kern


def _make_deg_pass(n_pad, e_pad):
    ept = e_pad // NW
    nb = ept // (CH * K)
    rpt = n_pad // NS

    @functools.partial(
        pl.kernel,
        out_type=jax.ShapeDtypeStruct((NC, n_pad), jnp.float32),
        mesh=_mesh(),
        scratch_types=[
            pltpu.VMEM((2, K, CH), jnp.int32),
            pltpu.VMEM((K, CH), jnp.float32),
            pltpu.VMEM_SHARED((n_pad,), jnp.float32),
            pltpu.SemaphoreType.DMA,
            pltpu.SemaphoreType.DMA,
        ],
        compiler_params=_SC_PARAMS,
    )
    def kern(dst_hbm, ones_hbm, zeros_hbm, out_hbm, didx, rows, agg_s,
             isem, ssem):
        cid = lax.axis_index("c")
        sid = lax.axis_index("s")
        wid = sid * NC + cid
        r0 = sid * rpt
        pltpu.sync_copy(zeros_hbm.at[pl.ds(r0, rpt)], agg_s.at[pl.ds(r0, rpt)])
        pltpu.sync_copy(ones_hbm, rows)
        plsc.subcore_barrier()

        row_base = wid * (ept // CH)

        def fire_idx(ro, s):
            pltpu.async_copy(dst_hbm.at[pl.ds(ro, K)], didx.at[s], isem)

        def drain_idx(s):
            pltpu.make_async_copy(dst_hbm.at[pl.ds(0, K)], didx.at[s],
                                  isem).wait()

        fire_idx(row_base, 0)

        def body(c, carry):
            s = c % 2
            drain_idx(s)
            ro_next = jnp.where(c + 1 < nb, row_base + (c + 1) * K, row_base)
            fire_idx(ro_next, (c + 1) % 2)
            sh = [pltpu.async_copy(rows.at[j], agg_s.at[didx.at[s, j]], ssem,
                                   add=True)
                  for j in range(K)]
            for h in sh:
                h.wait()
            return carry

        lax.fori_loop(0, nb, body, 0)
        drain_idx(nb % 2)
        plsc.subcore_barrier()
        pltpu.sync_copy(agg_s.at[pl.ds(r0, rpt)],
                        out_hbm.at[cid, pl.ds(r0, rpt)])

    return kern


def _tc_call(fn, ins):
    def body(*refs):
        in_refs = refs[:len(ins)]
        out_refs = refs[len(ins):]
        res = fn(*[r[...] for r in in_refs])
        for o_ref, o in zip(out_refs, res):
            o_ref[...] = o

    shapes = jax.eval_shape(lambda *a: fn(*a), *ins)
    return pl.pallas_call(
        body,
        out_shape=[jax.ShapeDtypeStruct(s.shape, s.dtype) for s in shapes],
    )(*ins)


def _bf(v):
    return v.astype(jnp.bfloat16).astype(jnp.float32)


def _plane_matmul(planes, W):
    din, dout = W.shape
    out = []
    for j in range(dout):
        acc = planes[0] * W[0, j]
        for i in range(1, din):
            acc = acc + planes[i] * W[i, j]
        out.append(acc)
    return out


def kernel(x, edge_index, m1_W1, m1_b1, m1_W2, m1_b2, m1_W3, m1_b3,
           m2_W1, m2_b1, m2_W2, m2_b2, m2_W3, m2_b3):
    n = x.shape[0]
    e = edge_index.shape[1]
    n_pad = _round_up(n + 1, 128)
    e_pad = _round_up(e, NW * CH * K)

    src = edge_index[0]
    dst = edge_index[1]
    pad_e = e_pad - e
    dummy = jnp.full((pad_e,), n, dtype=jnp.int32)
    src_r = jnp.concatenate([src, dummy]).reshape(e_pad // CH, CH)
    dst_r = jnp.concatenate([dst, dummy]).reshape(e_pad // CH, CH)

    rows128 = n_pad // 128
    x_plane = jnp.pad(x[:, 0], (0, n_pad - n)).reshape(rows128, 128)

    z1 = jnp.zeros((n_pad,), jnp.float32)
    ones = jnp.ones((K, CH), jnp.float32)

    b1_1 = m1_b1.reshape(1, 4)
    b2_1 = m1_b2.reshape(1, 4)
    b3_1 = m1_b3.reshape(1, 1)
    b1_2 = m2_b1.reshape(1, 4)
    b2_2 = m2_b2.reshape(1, 4)
    b3_2 = m2_b3.reshape(1, 1)

    deg_pass = _make_deg_pass(n_pad, e_pad)
    pass1 = _make_edge_pass1(n_pad, e_pad)
    pass4 = _make_edge_pass4(n_pad, e_pad)

    def plane(v):
        return v.reshape(rows128, 128)

    dcnt = deg_pass(dst_r, ones, z1)

    def e0(d0, d1, xp):
        deg = d0 + d1 + 1.0
        dinv = lax.rsqrt(deg)
        t1 = 10.0 * xp * dinv
        return (dinv, t1)
    dinv_p, t1_p = _tc_call(e0, [plane(dcnt[0]), plane(dcnt[1]), x_plane])

    def gcn_stage(t1_p, W1, b1, W2, b2, W3, b3):
        s = pass1(src_r, dst_r, t1_p.reshape(n_pad), z1)

        def e1(s0, s1, t1, dinv, W1_, b1_, W2_):
            u1 = dinv * (s0 + s1 + t1)
            o1 = [jnp.maximum(u1 * W1_[0, j] + b1_[0, j], 0.0)
                  for j in range(4)]
            h2 = _plane_matmul([_bf(o) for o in o1], _bf(W2_))
            g2 = [h * dinv for h in h2]
            return tuple(g2)
        g2 = _tc_call(e1, [plane(s[0]), plane(s[1]), t1_p, dinv_p, W1, b1, W2])

        a = pass4(src_r, dst_r, g2[0].reshape(n_pad), g2[1].reshape(n_pad),
                  g2[2].reshape(n_pad), g2[3].reshape(n_pad), z1)

        def e2(a0_0, a0_1, a0_2, a0_3, a1_0, a1_1, a1_2, a1_3,
               g0, g1, g2c, g3, dinv, b2_, W3_):
            ag0 = [a0_0, a0_1, a0_2, a0_3]
            ag1 = [a1_0, a1_1, a1_2, a1_3]
            g = [g0, g1, g2c, g3]
            o2 = [jnp.maximum(dinv * (ag0[j] + ag1[j] + g[j]) + b2_[0, j], 0.0)
                  for j in range(4)]
            W3b = _bf(W3_)
            h3 = _bf(o2[0]) * W3b[0, 0]
            for i in range(1, 4):
                h3 = h3 + _bf(o2[i]) * W3b[i, 0]
            t3 = h3 * dinv
            return (t3,)
        apl = [a[c, p].reshape(rows128, 128) for c in range(2)
               for p in range(4)]
        (t3_p,) = _tc_call(e2, apl + list(g2) +
                           [dinv_p, b2, W3])

        s3 = pass1(src_r, dst_r, t3_p.reshape(n_pad), z1)
        return s3, t3_p

    s3, t3_p = gcn_stage(t1_p, m1_W1, b1_1, m1_W2, b2_1, m1_W3, b3_1)

    def e3(s0, s1, t3, dinv, b3_):
        o3 = dinv * (s0 + s1 + t3) + b3_[0, 0]
        x2 = jnp.sin(o3 * 10.0)
        t1b = x2 * dinv
        return (x2, t1b)
    x2_p, t1b_p = _tc_call(e3, [plane(s3[0]), plane(s3[1]), t3_p, dinv_p,
                                b3_1])

    s3b, t3b_p = gcn_stage(t1b_p, m2_W1, b1_2, m2_W2, b2_2, m2_W3, b3_2)

    def e6(s0, s1, t3, dinv, b3_, x2):
        x3 = dinv * (s0 + s1 + t3) + b3_[0, 0]
        r = lax.broadcasted_iota(jnp.int32, x3.shape, 0)
        c = lax.broadcasted_iota(jnp.int32, x3.shape, 1)
        mask = (r * 128 + c) < n
        sx2 = jnp.sum(jnp.where(mask, x2, 0.0)).reshape(1, 1)
        sx3 = jnp.sum(jnp.where(mask, x3, 0.0)).reshape(1, 1)
        return (x3, sx2, sx3)
    x3_p, sx2, sx3 = _tc_call(e6, [plane(s3b[0]), plane(s3b[1]), t3b_p,
                                   dinv_p, b3_2, x2_p])

    x2_col = x2_p.reshape(n_pad)[:n]
    x3_col = x3_p.reshape(n_pad)[:n]
    node_emb = jnp.stack([x2_col, x3_col], axis=1)
    graph_emb = jnp.concatenate([sx2[0], sx3[0]]) / n
    return (graph_emb, node_emb)
